# baseline (speedup 1.0000x reference)
import functools

import jax
import jax.numpy as jnp
from jax import lax
from jax.experimental import pallas as pl
from jax.experimental.pallas import tpu as pltpu
from jax.experimental.pallas import tpu_sc as plsc

Q = 1024
DIM = 128
N_KEYS = 100000
TOPK = 16

S = 128
WPAD = 100352
G = WPAD // S
GPAD = 896

BQ1, BK1 = 512, 1024
QI, KI = Q // BQ1, WPAD // BK1
GB = BK1 // S

BQ2 = 256
BQ4 = 256

NEG = -1e30
INT_BIG = 2**30

NC, NS = 2, 16
NW = NC * NS
B_ROWS = Q * TOPK
CHUNK = 128
NCHUNK = B_ROWS // (NW * CHUNK)


def _matmul_body(q_ref, k_ref, sims_ref, gmax_ref):
    ki = pl.program_id(1)
    q = q_ref[...]
    nrm = jnp.sqrt(jnp.sum(q * q, axis=1, keepdims=True))
    qn = q / jnp.maximum(nrm, 1e-12)
    scores = lax.dot_general(qn, k_ref[...], (((1,), (1,)), ((), ())),
                             preferred_element_type=jnp.float32)
    col = ki * BK1 + lax.broadcasted_iota(jnp.int32, scores.shape, 1)
    scores = jnp.where(col < N_KEYS, scores, NEG)
    sims_ref[...] = scores
    parts = [jnp.max(scores[:, j * S:(j + 1) * S], axis=1, keepdims=True)
             for j in range(GB)]
    gmax_ref[0] = jnp.concatenate(parts, axis=1)


def _select_groups_body(gm_ref, gids_ref):
    g = gm_ref[...]
    giota = lax.broadcasted_iota(jnp.int32, g.shape, 1)
    cols = []
    for _ in range(TOPK):
        m = jnp.max(g, axis=1, keepdims=True)
        sel = jnp.min(jnp.where(g == m, giota, INT_BIG), axis=1, keepdims=True)
        cols.append(sel)
        g = jnp.where(giota == sel, NEG, g)
    gids_ref[...] = jnp.concatenate(cols, axis=1)


def _final_body(cand_ref, gids_ref, kd_ref, d_ref, i_ref, p_ref):
    vals = cand_ref[...]
    gids = gids_ref[...]
    l_iota = lax.broadcasted_iota(jnp.int32, (BQ4, S), 1)
    kidx = jnp.concatenate(
        [gids[:, j:j + 1] * S + l_iota for j in range(TOPK)], axis=1)
    dcols, icols = [], []
    for _ in range(TOPK):
        m = jnp.max(vals, axis=1, keepdims=True)
        sel = jnp.min(jnp.where(vals == m, kidx, INT_BIG), axis=1,
                      keepdims=True)
        dcols.append(m)
        icols.append(sel)
        vals = jnp.where(kidx == sel, NEG, vals)
    d = jnp.concatenate(dcols, axis=1) + kd_ref[0:1, 0:1]
    i = jnp.concatenate(icols, axis=1)
    nrm = jnp.sqrt(jnp.sum(d * d, axis=1, keepdims=True))
    dn = d / jnp.maximum(nrm, 1e-12)
    e = jnp.exp(dn - jnp.max(dn, axis=1, keepdims=True))
    d_ref[...] = d
    i_ref[...] = i
    p_ref[...] = e / jnp.sum(e, axis=1, keepdims=True)


@functools.cache
def _make_sc_gather():
    mesh = plsc.VectorSubcoreMesh(core_axis_name="c", subcore_axis_name="s")

    @functools.partial(
        pl.kernel,
        mesh=mesh,
        out_type=jax.ShapeDtypeStruct((NW, NCHUNK, CHUNK, S), jnp.float32),
        scratch_types=[
            pltpu.VMEM((NCHUNK, CHUNK), jnp.int32),
            pltpu.VMEM((NCHUNK, CHUNK, S), jnp.float32),
            pltpu.SemaphoreType.DMA,
        ],
    )
    def _sc_gather(table_ref, idx_ref, out_ref, idx_v, rows_v, sem):
        wid = lax.axis_index("s") * NC + lax.axis_index("c")
        pltpu.sync_copy(idx_ref.at[wid], idx_v)
        copies = [pltpu.async_copy(table_ref.at[idx_v.at[j]], rows_v.at[j],
                                   sem)
                  for j in range(1)]
        for c in copies:
            c.wait()
        pltpu.sync_copy(rows_v, out_ref.at[wid])

    return _sc_gather


def kernel(queries, keys, k):
    keys_p = jnp.pad(keys, ((0, WPAD - N_KEYS), (0, 0)))

    sims, gmax3 = pl.pallas_call(
        _matmul_body,
        grid=(QI, KI),
        in_specs=[
            pl.BlockSpec((BQ1, DIM), lambda qi, ki: (qi, 0)),
            pl.BlockSpec((BK1, DIM), lambda qi, ki: (ki, 0)),
        ],
        out_specs=[
            pl.BlockSpec((BQ1, BK1), lambda qi, ki: (qi, ki)),
            pl.BlockSpec((1, BQ1, GB), lambda qi, ki: (ki, qi, 0)),
        ],
        out_shape=[
            jax.ShapeDtypeStruct((Q, WPAD), jnp.float32),
            jax.ShapeDtypeStruct((KI, Q, GB), jnp.float32),
        ],
    )(queries, keys_p)

    gm = gmax3.transpose(1, 0, 2).reshape(Q, G)
    gm = jnp.pad(gm, ((0, 0), (0, GPAD - G)), constant_values=float(NEG))

    gids = pl.pallas_call(
        _select_groups_body,
        grid=(Q // BQ2,),
        in_specs=[pl.BlockSpec((BQ2, GPAD), lambda i: (i, 0))],
        out_specs=pl.BlockSpec((BQ2, TOPK), lambda i: (i, 0)),
        out_shape=jax.ShapeDtypeStruct((Q, TOPK), jnp.int32),
    )(gm)

    rowidx = jnp.arange(Q, dtype=jnp.int32)[:, None] * G + gids
    idx3 = rowidx.reshape(NW, NCHUNK, CHUNK)
    table = sims.reshape(Q * G, S)
    cand = _make_sc_gather()(table, idx3).reshape(Q, TOPK * S)

    return cand[:, :16], cand[:, 16:32], gids
    kd = jnp.full((8, 128), jnp.asarray(k - TOPK, jnp.float32))
    d, i, p = pl.pallas_call(
        _final_body,
        grid=(Q // BQ4,),
        in_specs=[
            pl.BlockSpec((BQ4, TOPK * S), lambda i: (i, 0)),
            pl.BlockSpec((BQ4, TOPK), lambda i: (i, 0)),
            pl.BlockSpec((8, 128), lambda i: (0, 0)),
        ],
        out_specs=[
            pl.BlockSpec((BQ4, TOPK), lambda i: (i, 0)),
            pl.BlockSpec((BQ4, TOPK), lambda i: (i, 0)),
            pl.BlockSpec((BQ4, TOPK), lambda i: (i, 0)),
        ],
        out_shape=[
            jax.ShapeDtypeStruct((Q, TOPK), jnp.float32),
            jax.ShapeDtypeStruct((Q, TOPK), jnp.int32),
            jax.ShapeDtypeStruct((Q, TOPK), jnp.float32),
        ],
    )(cand, gids, kd)
    return d, i, p


# speedup vs baseline: 1.3056x; 1.3056x over previous
import functools

import jax
import jax.numpy as jnp
from jax import lax
from jax.experimental import pallas as pl
from jax.experimental.pallas import tpu as pltpu
from jax.experimental.pallas import tpu_sc as plsc

Q = 1024
DIM = 128
N_KEYS = 100000
TOPK = 16

S = 128
WPAD = 100352
G = WPAD // S
GPAD = 896

BQ1, BK1 = 512, 1024
QI, KI = Q // BQ1, WPAD // BK1
GB = BK1 // S

BQ2 = 256
BQ4 = 256

NEG = -1e30
INT_BIG = 2**30

NC, NS = 2, 16
NW = NC * NS
B_ROWS = Q * TOPK
CHUNK = 128
NCHUNK = B_ROWS // (NW * CHUNK)


def _matmul_body(q_ref, k_ref, sims_ref, gmax_ref):
    ki = pl.program_id(1)
    q = q_ref[...]
    nrm = jnp.sqrt(jnp.sum(q * q, axis=1, keepdims=True))
    qn = q / jnp.maximum(nrm, 1e-12)
    scores = lax.dot_general(qn, k_ref[...], (((1,), (1,)), ((), ())),
                             preferred_element_type=jnp.float32)
    col = ki * BK1 + lax.broadcasted_iota(jnp.int32, scores.shape, 1)
    scores = jnp.where(col < N_KEYS, scores, NEG)
    gparts = []
    for j in range(GB):
        blk = scores[:, j * S:(j + 1) * S]
        sims_ref[:, j, :] = blk
        gparts.append(jnp.max(blk, axis=1, keepdims=True))
    gmax_ref[0] = jnp.concatenate(gparts, axis=1)


def _select_groups_body(gm_ref, gids_ref):
    g = gm_ref[...]
    giota = lax.broadcasted_iota(jnp.int32, g.shape, 1)
    cols = []
    for _ in range(TOPK):
        m = jnp.max(g, axis=1, keepdims=True)
        sel = jnp.min(jnp.where(g == m, giota, INT_BIG), axis=1, keepdims=True)
        cols.append(sel)
        g = jnp.where(giota == sel, NEG, g)
    gids_ref[...] = jnp.concatenate(cols, axis=1)


def _final_body(cand_ref, gids_ref, kd_ref, d_ref, i_ref, p_ref):
    vals = cand_ref[...]
    gids = gids_ref[...]
    l_iota = lax.broadcasted_iota(jnp.int32, (BQ4, S), 1)
    kidx = jnp.concatenate(
        [gids[:, j:j + 1] * S + l_iota for j in range(TOPK)], axis=1)
    dcols, icols = [], []
    for _ in range(TOPK):
        m = jnp.max(vals, axis=1, keepdims=True)
        sel = jnp.min(jnp.where(vals == m, kidx, INT_BIG), axis=1,
                      keepdims=True)
        dcols.append(m)
        icols.append(sel)
        vals = jnp.where(kidx == sel, NEG, vals)
    d = jnp.concatenate(dcols, axis=1) + kd_ref[0:1, 0:1]
    i = jnp.concatenate(icols, axis=1)
    nrm = jnp.sqrt(jnp.sum(d * d, axis=1, keepdims=True))
    dn = d / jnp.maximum(nrm, 1e-12)
    e = jnp.exp(dn - jnp.max(dn, axis=1, keepdims=True))
    d_ref[...] = d
    i_ref[...] = i
    p_ref[...] = e / jnp.sum(e, axis=1, keepdims=True)


@functools.cache
def _make_sc_gather():
    mesh = plsc.VectorSubcoreMesh(core_axis_name="c", subcore_axis_name="s")

    @functools.partial(
        pl.kernel,
        mesh=mesh,
        out_type=jax.ShapeDtypeStruct((NW, NCHUNK, CHUNK, S), jnp.float32),
        scratch_types=[
            pltpu.VMEM((NCHUNK, CHUNK), jnp.int32),
            pltpu.VMEM((NCHUNK, CHUNK, S), jnp.float32),
            pltpu.SemaphoreType.DMA,
        ],
    )
    def _sc_gather(table_ref, idx_ref, out_ref, idx_v, rows_v, sem):
        wid = lax.axis_index("s") * NC + lax.axis_index("c")
        pltpu.sync_copy(idx_ref.at[wid], idx_v)
        copies = [pltpu.async_copy(table_ref.at[idx_v.at[j]], rows_v.at[j],
                                   sem)
                  for j in range(NCHUNK)]
        for c in copies:
            c.wait()
        pltpu.sync_copy(rows_v, out_ref.at[wid])

    return _sc_gather


def kernel(queries, keys, k):
    sims, gmax3 = pl.pallas_call(
        _matmul_body,
        grid=(QI, KI),
        in_specs=[
            pl.BlockSpec((BQ1, DIM), lambda qi, ki: (qi, 0)),
            pl.BlockSpec((BK1, DIM), lambda qi, ki: (ki, 0)),
        ],
        out_specs=[
            pl.BlockSpec((BQ1, GB, S), lambda qi, ki: (qi, ki, 0)),
            pl.BlockSpec((1, BQ1, GB), lambda qi, ki: (ki, qi, 0)),
        ],
        out_shape=[
            jax.ShapeDtypeStruct((Q, G, S), jnp.float32),
            jax.ShapeDtypeStruct((KI, Q, GB), jnp.float32),
        ],
    )(queries, keys)

    gm = gmax3.transpose(1, 0, 2).reshape(Q, G)
    gm = jnp.pad(gm, ((0, 0), (0, GPAD - G)), constant_values=float(NEG))

    gids = pl.pallas_call(
        _select_groups_body,
        grid=(Q // BQ2,),
        in_specs=[pl.BlockSpec((BQ2, GPAD), lambda i: (i, 0))],
        out_specs=pl.BlockSpec((BQ2, TOPK), lambda i: (i, 0)),
        out_shape=jax.ShapeDtypeStruct((Q, TOPK), jnp.int32),
    )(gm)

    rowidx = jnp.arange(Q, dtype=jnp.int32)[:, None] * G + gids
    idx3 = rowidx.reshape(NW, NCHUNK, CHUNK)
    table = sims.reshape(Q * G, S)
    cand = _make_sc_gather()(table, idx3).reshape(Q, TOPK * S)

    kd = jnp.full((8, 128), jnp.asarray(k - TOPK, jnp.float32))
    d, i, p = pl.pallas_call(
        _final_body,
        grid=(Q // BQ4,),
        in_specs=[
            pl.BlockSpec((BQ4, TOPK * S), lambda i: (i, 0)),
            pl.BlockSpec((BQ4, TOPK), lambda i: (i, 0)),
            pl.BlockSpec((8, 128), lambda i: (0, 0)),
        ],
        out_specs=[
            pl.BlockSpec((BQ4, TOPK), lambda i: (i, 0)),
            pl.BlockSpec((BQ4, TOPK), lambda i: (i, 0)),
            pl.BlockSpec((BQ4, TOPK), lambda i: (i, 0)),
        ],
        out_shape=[
            jax.ShapeDtypeStruct((Q, TOPK), jnp.float32),
            jax.ShapeDtypeStruct((Q, TOPK), jnp.int32),
            jax.ShapeDtypeStruct((Q, TOPK), jnp.float32),
        ],
    )(cand, gids, kd)
    return d, i, p


# speedup vs baseline: 1.6495x; 1.2634x over previous
import functools

import jax
import jax.numpy as jnp
from jax import lax
from jax.experimental import pallas as pl
from jax.experimental.pallas import tpu as pltpu
from jax.experimental.pallas import tpu_sc as plsc

Q = 1024
DIM = 128
N_KEYS = 100000
TOPK = 16

S = 128
WPAD = 100352
G = WPAD // S
GPAD = 896

BQ1, BK1 = 512, 1024
QI, KI = Q // BQ1, WPAD // BK1
GB = BK1 // S

BQ2 = 256
BQ4 = 256

NEG = -1e30
INT_BIG = 2**30

NC, NS = 2, 16
NW = NC * NS
B_ROWS = Q * TOPK
CHUNK = 128
NCHUNK = B_ROWS // (NW * CHUNK)


def _matmul_body(q_ref, k_ref, sims_ref, gmax_ref):
    ki = pl.program_id(1)
    q = q_ref[...]
    nrm = jnp.sqrt(jnp.sum(q * q, axis=1, keepdims=True))
    qn = q / jnp.maximum(nrm, 1e-12)
    scores = lax.dot_general(qn, k_ref[...], (((1,), (1,)), ((), ())),
                             preferred_element_type=jnp.float32)
    col = ki * BK1 + lax.broadcasted_iota(jnp.int32, scores.shape, 1)
    scores = jnp.where(col < N_KEYS, scores, NEG)
    gparts = []
    for j in range(GB):
        blk = scores[:, j * S:(j + 1) * S]
        sims_ref[j] = blk
        gparts.append(jnp.max(blk, axis=1, keepdims=True))
    gmax_ref[0] = jnp.concatenate(gparts, axis=1)


def _select_groups_body(gm_ref, gids_ref):
    g = gm_ref[...]
    giota = lax.broadcasted_iota(jnp.int32, g.shape, 1)
    cols = []
    for _ in range(TOPK):
        m = jnp.max(g, axis=1, keepdims=True)
        sel = jnp.min(jnp.where(g == m, giota, INT_BIG), axis=1, keepdims=True)
        cols.append(sel)
        g = jnp.where(giota == sel, NEG, g)
    gids_ref[...] = jnp.concatenate(cols, axis=1)


def _final_body(cand_ref, gids_ref, kd_ref, d_ref, i_ref, p_ref):
    vals = cand_ref[...]
    gids = gids_ref[...]
    l_iota = lax.broadcasted_iota(jnp.int32, (BQ4, S), 1)
    kidx = jnp.concatenate(
        [gids[:, j:j + 1] * S + l_iota for j in range(TOPK)], axis=1)
    dcols, icols = [], []
    for _ in range(TOPK):
        m = jnp.max(vals, axis=1, keepdims=True)
        sel = jnp.min(jnp.where(vals == m, kidx, INT_BIG), axis=1,
                      keepdims=True)
        dcols.append(m)
        icols.append(sel)
        vals = jnp.where(kidx == sel, NEG, vals)
    d = jnp.concatenate(dcols, axis=1) + kd_ref[0:1, 0:1]
    i = jnp.concatenate(icols, axis=1)
    nrm = jnp.sqrt(jnp.sum(d * d, axis=1, keepdims=True))
    dn = d / jnp.maximum(nrm, 1e-12)
    e = jnp.exp(dn - jnp.max(dn, axis=1, keepdims=True))
    d_ref[...] = d
    i_ref[...] = i
    p_ref[...] = e / jnp.sum(e, axis=1, keepdims=True)


@functools.cache
def _make_sc_gather():
    mesh = plsc.VectorSubcoreMesh(core_axis_name="c", subcore_axis_name="s")

    @functools.partial(
        pl.kernel,
        mesh=mesh,
        out_type=jax.ShapeDtypeStruct((NW, NCHUNK, CHUNK, S), jnp.float32),
        scratch_types=[
            pltpu.VMEM((NCHUNK, CHUNK), jnp.int32),
            pltpu.VMEM((NCHUNK, CHUNK, S), jnp.float32),
            pltpu.SemaphoreType.DMA,
        ],
    )
    def _sc_gather(table_ref, idx_ref, out_ref, idx_v, rows_v, sem):
        wid = lax.axis_index("s") * NC + lax.axis_index("c")
        pltpu.sync_copy(idx_ref.at[wid], idx_v)
        copies = [pltpu.async_copy(table_ref.at[idx_v.at[j]], rows_v.at[j],
                                   sem)
                  for j in range(NCHUNK)]
        for c in copies:
            c.wait()
        pltpu.sync_copy(rows_v, out_ref.at[wid])

    return _sc_gather


def kernel(queries, keys, k):
    sims, gmax3 = pl.pallas_call(
        _matmul_body,
        grid=(QI, KI),
        in_specs=[
            pl.BlockSpec((BQ1, DIM), lambda qi, ki: (qi, 0)),
            pl.BlockSpec((BK1, DIM), lambda qi, ki: (ki, 0)),
        ],
        out_specs=[
            pl.BlockSpec((GB, BQ1, S), lambda qi, ki: (ki, qi, 0)),
            pl.BlockSpec((1, BQ1, GB), lambda qi, ki: (ki, qi, 0)),
        ],
        out_shape=[
            jax.ShapeDtypeStruct((G, Q, S), jnp.float32),
            jax.ShapeDtypeStruct((KI, Q, GB), jnp.float32),
        ],
    )(queries, keys)

    gm = gmax3.transpose(1, 0, 2).reshape(Q, G)
    gm = jnp.pad(gm, ((0, 0), (0, GPAD - G)), constant_values=float(NEG))

    gids = pl.pallas_call(
        _select_groups_body,
        grid=(Q // BQ2,),
        in_specs=[pl.BlockSpec((BQ2, GPAD), lambda i: (i, 0))],
        out_specs=pl.BlockSpec((BQ2, TOPK), lambda i: (i, 0)),
        out_shape=jax.ShapeDtypeStruct((Q, TOPK), jnp.int32),
    )(gm)

    rowidx = gids * Q + jnp.arange(Q, dtype=jnp.int32)[:, None]
    idx3 = rowidx.reshape(NW, NCHUNK, CHUNK)
    table = sims.reshape(Q * G, S)
    cand = _make_sc_gather()(table, idx3).reshape(Q, TOPK * S)

    kd = jnp.full((8, 128), jnp.asarray(k - TOPK, jnp.float32))
    d, i, p = pl.pallas_call(
        _final_body,
        grid=(Q // BQ4,),
        in_specs=[
            pl.BlockSpec((BQ4, TOPK * S), lambda i: (i, 0)),
            pl.BlockSpec((BQ4, TOPK), lambda i: (i, 0)),
            pl.BlockSpec((8, 128), lambda i: (0, 0)),
        ],
        out_specs=[
            pl.BlockSpec((BQ4, TOPK), lambda i: (i, 0)),
            pl.BlockSpec((BQ4, TOPK), lambda i: (i, 0)),
            pl.BlockSpec((BQ4, TOPK), lambda i: (i, 0)),
        ],
        out_shape=[
            jax.ShapeDtypeStruct((Q, TOPK), jnp.float32),
            jax.ShapeDtypeStruct((Q, TOPK), jnp.int32),
            jax.ShapeDtypeStruct((Q, TOPK), jnp.float32),
        ],
    )(cand, gids, kd)
    return d, i, p


# speedup vs baseline: 2.7180x; 1.6478x over previous
import functools

import jax
import jax.numpy as jnp
from jax import lax
from jax.experimental import pallas as pl
from jax.experimental.pallas import tpu as pltpu
from jax.experimental.pallas import tpu_sc as plsc

Q = 1024
DIM = 128
N_KEYS = 100000
TOPK = 16

S = 128
WPAD = 102400
G = WPAD // S
GPAD = 896

BQ1, BK1 = 1024, 4096
QI, KI = Q // BQ1, WPAD // BK1
GB = BK1 // S

BQ4 = 256

NEG = -1e30
INT_BIG = 2**30

NC, NS = 2, 16
NW = NC * NS
B_ROWS = Q * TOPK
CHUNK = 128
NCHUNK = B_ROWS // (NW * CHUNK)


def _matmul_body(q_ref, k_ref, sims_ref, gids_ref, qn_s, gm_s):
    ki = pl.program_id(1)

    @pl.when(ki == 0)
    def _init():
        q = q_ref[...]
        nrm = jnp.sqrt(jnp.sum(q * q, axis=1, keepdims=True))
        qn_s[...] = q / jnp.maximum(nrm, 1e-12)
        gm_s[...] = jnp.full((GPAD, BQ1), NEG, jnp.float32)

    scores = lax.dot_general(qn_s[...], k_ref[...], (((1,), (1,)), ((), ())),
                             preferred_element_type=jnp.float32)
    col = ki * BK1 + lax.broadcasted_iota(jnp.int32, scores.shape, 1)
    scores = jnp.where(col < N_KEYS, scores, NEG)
    gparts = []
    for j in range(GB):
        blk = scores[:, j * S:(j + 1) * S]
        sims_ref[j] = blk
        gparts.append(jnp.max(blk, axis=1, keepdims=True))
    gm_s[pl.ds(ki * GB, GB), :] = jnp.concatenate(gparts, axis=1).T

    @pl.when(ki == KI - 1)
    def _select():
        g = gm_s[...]
        giota = lax.broadcasted_iota(jnp.int32, g.shape, 0)
        rows = []
        for _ in range(TOPK):
            m = jnp.max(g, axis=0, keepdims=True)
            sel = jnp.min(jnp.where(g == m, giota, INT_BIG), axis=0,
                          keepdims=True)
            rows.append(sel)
            g = jnp.where(giota == sel, NEG, g)
        gids_ref[...] = jnp.concatenate(rows, axis=0)


def _final_body(cand_ref, gids_ref, kd_ref, d_ref, i_ref, p_ref):
    vals = cand_ref[...]
    gids = gids_ref[...]
    l_iota = lax.broadcasted_iota(jnp.int32, (BQ4, S), 1)
    kidx = jnp.concatenate(
        [gids[:, j:j + 1] * S + l_iota for j in range(TOPK)], axis=1)
    dcols, icols = [], []
    for _ in range(TOPK):
        m = jnp.max(vals, axis=1, keepdims=True)
        sel = jnp.min(jnp.where(vals == m, kidx, INT_BIG), axis=1,
                      keepdims=True)
        dcols.append(m)
        icols.append(sel)
        vals = jnp.where(kidx == sel, NEG, vals)
    d = jnp.concatenate(dcols, axis=1) + kd_ref[0:1, 0:1]
    i = jnp.concatenate(icols, axis=1)
    nrm = jnp.sqrt(jnp.sum(d * d, axis=1, keepdims=True))
    dn = d / jnp.maximum(nrm, 1e-12)
    e = jnp.exp(dn - jnp.max(dn, axis=1, keepdims=True))
    d_ref[...] = d
    i_ref[...] = i
    p_ref[...] = e / jnp.sum(e, axis=1, keepdims=True)


@functools.cache
def _make_sc_gather():
    mesh = plsc.VectorSubcoreMesh(core_axis_name="c", subcore_axis_name="s")

    @functools.partial(
        pl.kernel,
        mesh=mesh,
        out_type=jax.ShapeDtypeStruct((NW, NCHUNK, CHUNK, S), jnp.float32),
        scratch_types=[
            pltpu.VMEM((NCHUNK, CHUNK), jnp.int32),
            pltpu.VMEM((NCHUNK, CHUNK, S), jnp.float32),
            pltpu.SemaphoreType.DMA,
        ],
    )
    def _sc_gather(table_ref, idx_ref, out_ref, idx_v, rows_v, sem):
        wid = lax.axis_index("s") * NC + lax.axis_index("c")
        pltpu.sync_copy(idx_ref.at[wid], idx_v)
        copies = [pltpu.async_copy(table_ref.at[idx_v.at[j]], rows_v.at[j],
                                   sem)
                  for j in range(NCHUNK)]
        for c in copies:
            c.wait()
        pltpu.sync_copy(rows_v, out_ref.at[wid])

    return _sc_gather


def kernel(queries, keys, k):
    sims, gidsT = pl.pallas_call(
        _matmul_body,
        grid=(QI, KI),
        in_specs=[
            pl.BlockSpec((BQ1, DIM), lambda qi, ki: (qi, 0)),
            pl.BlockSpec((BK1, DIM), lambda qi, ki: (ki, 0)),
        ],
        out_specs=[
            pl.BlockSpec((GB, BQ1, S), lambda qi, ki: (ki, qi, 0)),
            pl.BlockSpec((TOPK, BQ1), lambda qi, ki: (0, qi)),
        ],
        out_shape=[
            jax.ShapeDtypeStruct((G, Q, S), jnp.float32),
            jax.ShapeDtypeStruct((TOPK, Q), jnp.int32),
        ],
        scratch_shapes=[
            pltpu.VMEM((BQ1, DIM), jnp.float32),
            pltpu.VMEM((GPAD, BQ1), jnp.float32),
        ],
    )(queries, keys)
    gids = gidsT.T

    rowidx = gids * Q + jnp.arange(Q, dtype=jnp.int32)[:, None]
    idx3 = rowidx.reshape(NW, NCHUNK, CHUNK)
    table = sims.reshape(Q * G, S)
    cand = _make_sc_gather()(table, idx3).reshape(Q, TOPK * S)

    kd = jnp.full((8, 128), jnp.asarray(k - TOPK, jnp.float32))
    d, i, p = pl.pallas_call(
        _final_body,
        grid=(Q // BQ4,),
        in_specs=[
            pl.BlockSpec((BQ4, TOPK * S), lambda i: (i, 0)),
            pl.BlockSpec((BQ4, TOPK), lambda i: (i, 0)),
            pl.BlockSpec((8, 128), lambda i: (0, 0)),
        ],
        out_specs=[
            pl.BlockSpec((BQ4, TOPK), lambda i: (i, 0)),
            pl.BlockSpec((BQ4, TOPK), lambda i: (i, 0)),
            pl.BlockSpec((BQ4, TOPK), lambda i: (i, 0)),
        ],
        out_shape=[
            jax.ShapeDtypeStruct((Q, TOPK), jnp.float32),
            jax.ShapeDtypeStruct((Q, TOPK), jnp.int32),
            jax.ShapeDtypeStruct((Q, TOPK), jnp.float32),
        ],
    )(cand, gids, kd)
    return d, i, p


# speedup vs baseline: 2.7307x; 1.0047x over previous
import functools

import jax
import jax.numpy as jnp
from jax import lax
from jax.experimental import pallas as pl
from jax.experimental.pallas import tpu as pltpu
from jax.experimental.pallas import tpu_sc as plsc

Q = 1024
DIM = 128
N_KEYS = 100000
TOPK = 16

S = 128
WPAD = 102400
G = WPAD // S
GPAD = 896

BQ1, BK1 = 1024, 5120
QI, KI = Q // BQ1, WPAD // BK1
GB = BK1 // S

BQ4 = 256

NEG = -1e30
INT_BIG = 2**30

NC, NS = 2, 16
NW = NC * NS
B_ROWS = Q * TOPK
CHUNK = 128
NCHUNK = B_ROWS // (NW * CHUNK)


def _matmul_body(q_ref, k_ref, sims_ref, gids_ref, qn_s, gm_s):
    ki = pl.program_id(1)

    @pl.when(ki == 0)
    def _init():
        q = q_ref[...]
        nrm = jnp.sqrt(jnp.sum(q * q, axis=1, keepdims=True))
        qn_s[...] = q / jnp.maximum(nrm, 1e-12)
        gm_s[...] = jnp.full((GPAD, BQ1), NEG, jnp.float32)

    scores = lax.dot_general(qn_s[...], k_ref[...], (((1,), (1,)), ((), ())),
                             preferred_element_type=jnp.float32)
    col = ki * BK1 + lax.broadcasted_iota(jnp.int32, scores.shape, 1)
    scores = jnp.where(col < N_KEYS, scores, NEG)
    gparts = []
    for j in range(GB):
        blk = scores[:, j * S:(j + 1) * S]
        sims_ref[j] = blk
        gparts.append(jnp.max(blk, axis=1, keepdims=True))
    gm_s[pl.ds(ki * GB, GB), :] = jnp.concatenate(gparts, axis=1).T

    @pl.when(ki == KI - 1)
    def _select():
        g = gm_s[...]
        giota = lax.broadcasted_iota(jnp.int32, g.shape, 0)
        rows = []
        for _ in range(TOPK):
            m = jnp.max(g, axis=0, keepdims=True)
            sel = jnp.min(jnp.where(g == m, giota, INT_BIG), axis=0,
                          keepdims=True)
            rows.append(sel)
            g = jnp.where(giota == sel, NEG, g)
        gids_ref[...] = jnp.concatenate(rows, axis=0)


def _final_body(cand_ref, gids_ref, kd_ref, d_ref, i_ref, p_ref):
    vals = cand_ref[...]
    gids = gids_ref[...]
    l_iota = lax.broadcasted_iota(jnp.int32, (BQ4, S), 1)
    kidx = jnp.concatenate(
        [gids[:, j:j + 1] * S + l_iota for j in range(TOPK)], axis=1)
    dcols, icols = [], []
    for _ in range(TOPK):
        m = jnp.max(vals, axis=1, keepdims=True)
        sel = jnp.min(jnp.where(vals == m, kidx, INT_BIG), axis=1,
                      keepdims=True)
        dcols.append(m)
        icols.append(sel)
        vals = jnp.where(kidx == sel, NEG, vals)
    d = jnp.concatenate(dcols, axis=1) + kd_ref[0:1, 0:1]
    i = jnp.concatenate(icols, axis=1)
    nrm = jnp.sqrt(jnp.sum(d * d, axis=1, keepdims=True))
    dn = d / jnp.maximum(nrm, 1e-12)
    e = jnp.exp(dn - jnp.max(dn, axis=1, keepdims=True))
    d_ref[...] = d
    i_ref[...] = i
    p_ref[...] = e / jnp.sum(e, axis=1, keepdims=True)


@functools.cache
def _make_sc_gather():
    mesh = plsc.VectorSubcoreMesh(core_axis_name="c", subcore_axis_name="s")

    @functools.partial(
        pl.kernel,
        mesh=mesh,
        out_type=jax.ShapeDtypeStruct((NW, NCHUNK, CHUNK, S), jnp.float32),
        scratch_types=[
            pltpu.VMEM((NCHUNK, CHUNK), jnp.int32),
            pltpu.VMEM((NCHUNK, CHUNK, S), jnp.float32),
            pltpu.SemaphoreType.DMA,
        ],
    )
    def _sc_gather(table_ref, idx_ref, out_ref, idx_v, rows_v, sem):
        wid = lax.axis_index("s") * NC + lax.axis_index("c")
        pltpu.sync_copy(idx_ref.at[wid], idx_v)
        copies = [pltpu.async_copy(table_ref.at[idx_v.at[j]], rows_v.at[j],
                                   sem)
                  for j in range(NCHUNK)]
        for c in copies:
            c.wait()
        pltpu.sync_copy(rows_v, out_ref.at[wid])

    return _sc_gather


def kernel(queries, keys, k):
    sims, gidsT = pl.pallas_call(
        _matmul_body,
        grid=(QI, KI),
        in_specs=[
            pl.BlockSpec((BQ1, DIM), lambda qi, ki: (qi, 0)),
            pl.BlockSpec((BK1, DIM), lambda qi, ki: (ki, 0)),
        ],
        out_specs=[
            pl.BlockSpec((GB, BQ1, S), lambda qi, ki: (ki, qi, 0)),
            pl.BlockSpec((TOPK, BQ1), lambda qi, ki: (0, qi)),
        ],
        out_shape=[
            jax.ShapeDtypeStruct((G, Q, S), jnp.float32),
            jax.ShapeDtypeStruct((TOPK, Q), jnp.int32),
        ],
        scratch_shapes=[
            pltpu.VMEM((BQ1, DIM), jnp.float32),
            pltpu.VMEM((GPAD, BQ1), jnp.float32),
        ],
    )(queries, keys)
    gids = gidsT.T

    rowidx = gids * Q + jnp.arange(Q, dtype=jnp.int32)[:, None]
    idx3 = rowidx.reshape(NW, NCHUNK, CHUNK)
    table = sims.reshape(Q * G, S)
    cand = _make_sc_gather()(table, idx3).reshape(Q, TOPK * S)

    kd = jnp.full((8, 128), jnp.asarray(k - TOPK, jnp.float32))
    d, i, p = pl.pallas_call(
        _final_body,
        grid=(Q // BQ4,),
        in_specs=[
            pl.BlockSpec((BQ4, TOPK * S), lambda i: (i, 0)),
            pl.BlockSpec((BQ4, TOPK), lambda i: (i, 0)),
            pl.BlockSpec((8, 128), lambda i: (0, 0)),
        ],
        out_specs=[
            pl.BlockSpec((BQ4, TOPK), lambda i: (i, 0)),
            pl.BlockSpec((BQ4, TOPK), lambda i: (i, 0)),
            pl.BlockSpec((BQ4, TOPK), lambda i: (i, 0)),
        ],
        out_shape=[
            jax.ShapeDtypeStruct((Q, TOPK), jnp.float32),
            jax.ShapeDtypeStruct((Q, TOPK), jnp.int32),
            jax.ShapeDtypeStruct((Q, TOPK), jnp.float32),
        ],
    )(cand, gids, kd)
    return d, i, p
